# baseline (speedup 1.0000x reference)
import functools

import jax
import jax.numpy as jnp
from jax import lax
from jax.experimental import pallas as pl
from jax.experimental.pallas import tpu as pltpu
from jax.experimental.pallas import tpu_sc as plsc

NUM_LATENT = 32
BATCH = 16384

_info = plsc.get_sparse_core_info()
_NC = _info.num_cores
_NS = _info.num_subcores
_NW = _NC * _NS
_BPW = BATCH // _NW
_CHUNK = 128
_NCHUNK = _BPW // _CHUNK
_GROUP = 16


def _sc_embed_dot(user_ids, movie_ids, user_table, movie_table):
    mesh = plsc.VectorSubcoreMesh(core_axis_name="c", subcore_axis_name="s")

    @functools.partial(
        pl.kernel,
        mesh=mesh,
        out_type=jax.ShapeDtypeStruct((BATCH,), jnp.float32),
        compiler_params=pltpu.CompilerParams(needs_layout_passes=False,
                                             use_tc_tiling_on_sc=False),
        scratch_types=[
            pltpu.VMEM((_NCHUNK, _CHUNK), jnp.int32),
            pltpu.VMEM((_NCHUNK, _CHUNK), jnp.int32),
            pltpu.VMEM((_BPW, NUM_LATENT), jnp.float32),
            pltpu.VMEM((_BPW, NUM_LATENT), jnp.float32),
            pltpu.VMEM((_BPW,), jnp.float32),
            pltpu.SemaphoreType.DMA,
        ],
    )
    def k(uid_hbm, mid_hbm, ut_hbm, mt_hbm, out_hbm,
          uid_v, mid_v, urows_v, mrows_v, out_v, sem):
        wid = lax.axis_index("s") * _NC + lax.axis_index("c")
        base = wid * _BPW

        for c in range(_NCHUNK):
            pltpu.sync_copy(uid_hbm.at[pl.ds(base + c * _CHUNK, _CHUNK)],
                            uid_v.at[c])
            pltpu.sync_copy(mid_hbm.at[pl.ds(base + c * _CHUNK, _CHUNK)],
                            mid_v.at[c])

        copies = []
        for c in range(_NCHUNK):
            copies.append(pltpu.async_copy(
                ut_hbm.at[uid_v.at[c]],
                urows_v.at[pl.ds(c * _CHUNK, _CHUNK)], sem))
            copies.append(pltpu.async_copy(
                mt_hbm.at[mid_v.at[c]],
                mrows_v.at[pl.ds(c * _CHUNK, _CHUNK)], sem))
        for cp in copies:
            cp.wait()

        lanes = lax.iota(jnp.int32, _GROUP)

        def body(g, carry):
            row_idx = lanes + g * _GROUP
            acc = jnp.zeros((_GROUP,), jnp.float32)
            for j in range(NUM_LATENT):
                col = jnp.full((_GROUP,), j, jnp.int32)
                u = plsc.load_gather(urows_v, [row_idx, col])
                m = plsc.load_gather(mrows_v, [row_idx, col])
                acc = acc + u * m
            out_v[pl.ds(g * _GROUP, _GROUP)] = acc
            return carry

        lax.fori_loop(0, _BPW // _GROUP, body, 0)

        pltpu.sync_copy(out_v, out_hbm.at[pl.ds(base, _BPW)])

    return k(user_ids, movie_ids, user_table, movie_table)


def kernel(user_ids, movie_ids, user_table, movie_table):
    return _sc_embed_dot(user_ids.astype(jnp.int32),
                         movie_ids.astype(jnp.int32),
                         user_table, movie_table)


# speedup vs baseline: 1.0076x; 1.0076x over previous
import functools

import jax
import jax.numpy as jnp
from jax import lax
from jax.experimental import pallas as pl
from jax.experimental.pallas import tpu as pltpu
from jax.experimental.pallas import tpu_sc as plsc

NUM_LATENT = 32
BATCH = 16384

_info = plsc.get_sparse_core_info()
_NC = _info.num_cores
_NS = _info.num_subcores
_NW = _NC * _NS
_BPW = BATCH // _NW
_CHUNK = 128
_NCHUNK = _BPW // _CHUNK
_GROUP = 16

_PARAMS = pltpu.CompilerParams(needs_layout_passes=False,
                               use_tc_tiling_on_sc=False)
_MESH = dict(core_axis_name="c", subcore_axis_name="s")


def _worker_base():
    wid = lax.axis_index("s") * _NC + lax.axis_index("c")
    return wid * _BPW


def _gather_rows(ids_hbm, table_hbm, ids_v, rows_v, sem, base):
    for c in range(_NCHUNK):
        pltpu.sync_copy(ids_hbm.at[pl.ds(base + c * _CHUNK, _CHUNK)],
                        ids_v.at[c])
    copies = []
    for c in range(_NCHUNK):
        copies.append(pltpu.async_copy(
            table_hbm.at[ids_v.at[c]],
            rows_v.at[pl.ds(c * _CHUNK, _CHUNK)], sem))
    return copies


def _movie_gather(movie_ids, movie_table):
    @functools.partial(
        pl.kernel,
        mesh=plsc.VectorSubcoreMesh(**_MESH),
        out_type=jax.ShapeDtypeStruct((BATCH, NUM_LATENT), jnp.float32),
        compiler_params=_PARAMS,
        scratch_types=[
            pltpu.VMEM((_NCHUNK, _CHUNK), jnp.int32),
            pltpu.VMEM((_BPW, NUM_LATENT), jnp.float32),
            pltpu.SemaphoreType.DMA,
        ],
    )
    def k(mid_hbm, mt_hbm, mrows_hbm, mid_v, mrows_v, sem):
        base = _worker_base()
        for cp in _gather_rows(mid_hbm, mt_hbm, mid_v, mrows_v, sem, base):
            cp.wait()
        pltpu.sync_copy(mrows_v, mrows_hbm.at[pl.ds(base, _BPW), :])

    return k(movie_ids, movie_table)


def _user_gather_dot(user_ids, user_table, mrows):
    @functools.partial(
        pl.kernel,
        mesh=plsc.VectorSubcoreMesh(**_MESH),
        out_type=jax.ShapeDtypeStruct((BATCH,), jnp.float32),
        compiler_params=_PARAMS,
        scratch_types=[
            pltpu.VMEM((_NCHUNK, _CHUNK), jnp.int32),
            pltpu.VMEM((_BPW, NUM_LATENT), jnp.float32),
            pltpu.VMEM((_BPW, NUM_LATENT), jnp.float32),
            pltpu.VMEM((_BPW,), jnp.float32),
            pltpu.SemaphoreType.DMA,
            pltpu.SemaphoreType.DMA,
        ],
    )
    def k(uid_hbm, ut_hbm, mrows_hbm, out_hbm,
          uid_v, urows_v, mrows_v, out_v, semg, semm):
        base = _worker_base()
        mcp = pltpu.async_copy(mrows_hbm.at[pl.ds(base, _BPW), :],
                               mrows_v, semm)
        copies = _gather_rows(uid_hbm, ut_hbm, uid_v, urows_v, semg, base)
        mcp.wait()
        for cp in copies:
            cp.wait()

        lanes = lax.iota(jnp.int32, _GROUP)

        def body(g, carry):
            row_idx = lanes + g * _GROUP
            acc = jnp.zeros((_GROUP,), jnp.float32)
            for j in range(NUM_LATENT):
                col = jnp.full((_GROUP,), j, jnp.int32)
                u = plsc.load_gather(urows_v, [row_idx, col])
                m = plsc.load_gather(mrows_v, [row_idx, col])
                acc = acc + u * m
            out_v[pl.ds(g * _GROUP, _GROUP)] = acc
            return carry

        lax.fori_loop(0, _BPW // _GROUP, body, 0)
        pltpu.sync_copy(out_v, out_hbm.at[pl.ds(base, _BPW)])

    return k(user_ids, user_table, mrows)


def kernel(user_ids, movie_ids, user_table, movie_table):
    mrows = _movie_gather(movie_ids.astype(jnp.int32), movie_table)
    return _user_gather_dot(user_ids.astype(jnp.int32), user_table, mrows)


# speedup vs baseline: 2.8667x; 2.8452x over previous
import functools

import jax
import jax.numpy as jnp
from jax import lax
from jax.experimental import pallas as pl
from jax.experimental.pallas import tpu as pltpu
from jax.experimental.pallas import tpu_sc as plsc

NUM_LATENT = 32
BATCH = 16384

_info = plsc.get_sparse_core_info()
_NC = _info.num_cores
_NS = _info.num_subcores
_NW = _NC * _NS
_BPW = BATCH // _NW
_CHUNK = 128
_NCHUNK = _BPW // _CHUNK
_G = 16
_SUPER = 32
_NSUPER = _BPW // _SUPER
_MROWS = _SUPER * NUM_LATENT // 128


def _movie_gather(movie_ids, movie_table):
    @functools.partial(
        pl.kernel,
        mesh=plsc.VectorSubcoreMesh(core_axis_name="c", subcore_axis_name="s"),
        out_type=jax.ShapeDtypeStruct((BATCH, NUM_LATENT), jnp.float32),
        compiler_params=pltpu.CompilerParams(needs_layout_passes=False,
                                             use_tc_tiling_on_sc=False),
        scratch_types=[
            pltpu.VMEM((_NCHUNK, _CHUNK), jnp.int32),
            pltpu.VMEM((_BPW, NUM_LATENT), jnp.float32),
            pltpu.SemaphoreType.DMA,
        ],
    )
    def k(mid_hbm, mt_hbm, mrows_hbm, mid_v, mrows_v, sem):
        wid = lax.axis_index("s") * _NC + lax.axis_index("c")
        base = wid * _BPW
        for c in range(_NCHUNK):
            pltpu.sync_copy(mid_hbm.at[pl.ds(base + c * _CHUNK, _CHUNK)],
                            mid_v.at[c])
        copies = []
        for c in range(_NCHUNK):
            copies.append(pltpu.async_copy(
                mt_hbm.at[mid_v.at[c]],
                mrows_v.at[pl.ds(c * _CHUNK, _CHUNK)], sem))
        for cp in copies:
            cp.wait()
        pltpu.sync_copy(mrows_v, mrows_hbm.at[pl.ds(base, _BPW), :])

    return k(movie_ids, movie_table)


def _user_slab_dot(user_ids, utT, m128):
    @functools.partial(
        pl.kernel,
        mesh=plsc.VectorSubcoreMesh(core_axis_name="c", subcore_axis_name="s"),
        out_type=jax.ShapeDtypeStruct((BATCH,), jnp.float32),
        compiler_params=pltpu.CompilerParams(needs_layout_passes=False),
        scratch_types=[
            pltpu.VMEM((_BPW,), jnp.int32),
            pltpu.VMEM((_G, NUM_LATENT, 128), jnp.float32),
            pltpu.VMEM((2, _MROWS, 128), jnp.float32),
            pltpu.VMEM((_BPW,), jnp.float32),
            pltpu.SemaphoreType.DMA,
            pltpu.SemaphoreType.DMA,
        ],
    )
    def k(uid_hbm, utT_hbm, m128_hbm, out_hbm,
          uid_v, slabs, mbuf, out_v, semu, semm):
        wid = lax.axis_index("s") * _NC + lax.axis_index("c")
        base = wid * _BPW
        mbase = wid * (_BPW * NUM_LATENT // 128)
        pltpu.sync_copy(uid_hbm.at[pl.ds(base, _BPW)], uid_v)
        lanes = lax.iota(jnp.int32, _G)

        pltpu.async_copy(m128_hbm.at[pl.ds(mbase, _MROWS), :],
                         mbuf.at[0], semm)

        def superblock(G, carry):
            @pl.when(G < _NSUPER - 1)
            def _():
                off = pl.multiple_of(mbase + (G + 1) * _MROWS, 8)
                pltpu.async_copy(m128_hbm.at[pl.ds(off, _MROWS), :],
                                 mbuf.at[(G + 1) & 1], semm)
            cur = pl.multiple_of(mbase + G * _MROWS, 8)
            pltpu.make_async_copy(m128_hbm.at[pl.ds(cur, _MROWS), :],
                                  mbuf.at[G & 1], semm).wait()
            gbuf = jnp.zeros((_G,), jnp.int32) + (G & 1)
            for h in range(_SUPER // _G):
                chunk = uid_v[pl.ds(G * _SUPER + h * _G, _G)]
                cps = []
                for e in range(_G):
                    u = chunk[e]
                    col0 = pl.multiple_of((u >> 7) * 128, 128)
                    cps.append(pltpu.async_copy(
                        utT_hbm.at[:, pl.ds(col0, 128)], slabs.at[e], semu))
                for cp in cps:
                    cp.wait()
                lane = chunk & 127
                mrow = (lanes >> 2) + h * (_G // 4)
                mcol0 = (lanes & 3) * NUM_LATENT
                acc = jnp.zeros((_G,), jnp.float32)
                for j in range(NUM_LATENT):
                    uj = plsc.load_gather(
                        slabs, [lanes, jnp.full((_G,), j, jnp.int32), lane])
                    mj = plsc.load_gather(mbuf, [gbuf, mrow, mcol0 + j])
                    acc = acc + uj * mj
                out_v[pl.ds(G * _SUPER + h * _G, _G)] = acc
            return carry

        lax.fori_loop(0, _NSUPER, superblock, 0)
        pltpu.sync_copy(out_v, out_hbm.at[pl.ds(base, _BPW)])

    return k(user_ids, utT, m128)


def kernel(user_ids, movie_ids, user_table, movie_table):
    mrows = _movie_gather(movie_ids.astype(jnp.int32), movie_table)
    m128 = mrows.reshape(BATCH * NUM_LATENT // 128, 128)
    return _user_slab_dot(user_ids.astype(jnp.int32), user_table.T, m128)


# speedup vs baseline: 2.9035x; 1.0128x over previous
import functools

import jax
import jax.numpy as jnp
from jax import lax
from jax.experimental import pallas as pl
from jax.experimental.pallas import tpu as pltpu
from jax.experimental.pallas import tpu_sc as plsc

NUM_LATENT = 32
BATCH = 16384

_info = plsc.get_sparse_core_info()
_NC = _info.num_cores
_NS = _info.num_subcores
_NW = _NC * _NS
_BPW = BATCH // _NW
_CHUNK = 128
_NCHUNK = _BPW // _CHUNK
_G = 16
_SUPER = 32
_NSUPER = _BPW // _SUPER
_MROWS = _SUPER * NUM_LATENT // 128


def _movie_gather(movie_ids, movie_table):
    @functools.partial(
        pl.kernel,
        mesh=plsc.VectorSubcoreMesh(core_axis_name="c", subcore_axis_name="s"),
        out_type=jax.ShapeDtypeStruct((BATCH, NUM_LATENT), jnp.float32),
        compiler_params=pltpu.CompilerParams(needs_layout_passes=False,
                                             use_tc_tiling_on_sc=False),
        scratch_types=[
            pltpu.VMEM((_NCHUNK, _CHUNK), jnp.int32),
            pltpu.VMEM((_BPW, NUM_LATENT), jnp.float32),
            pltpu.SemaphoreType.DMA,
        ],
    )
    def k(mid_hbm, mt_hbm, mrows_hbm, mid_v, mrows_v, sem):
        wid = lax.axis_index("s") * _NC + lax.axis_index("c")
        base = wid * _BPW
        for c in range(_NCHUNK):
            pltpu.sync_copy(mid_hbm.at[pl.ds(base + c * _CHUNK, _CHUNK)],
                            mid_v.at[c])
        copies = []
        for c in range(_NCHUNK):
            copies.append(pltpu.async_copy(
                mt_hbm.at[mid_v.at[c]],
                mrows_v.at[pl.ds(c * _CHUNK, _CHUNK)], sem))
        for cp in copies:
            cp.wait()
        pltpu.sync_copy(mrows_v, mrows_hbm.at[pl.ds(base, _BPW), :])

    return k(movie_ids, movie_table)


_W = 8
_NWAVE = _BPW // _W
_PAD = _BPW + _G


def _user_slab_dot(user_ids, utT, m128):
    @functools.partial(
        pl.kernel,
        mesh=plsc.VectorSubcoreMesh(core_axis_name="c", subcore_axis_name="s"),
        out_type=jax.ShapeDtypeStruct((BATCH,), jnp.float32),
        compiler_params=pltpu.CompilerParams(needs_layout_passes=False),
        scratch_types=[
            pltpu.VMEM((_PAD,), jnp.int32),
            pltpu.VMEM((2 * _W, NUM_LATENT, 128), jnp.float32),
            pltpu.VMEM((_BPW * NUM_LATENT // 128, 128), jnp.float32),
            pltpu.VMEM((_PAD,), jnp.float32),
            pltpu.SemaphoreType.DMA,
            pltpu.SemaphoreType.DMA,
            pltpu.SemaphoreType.DMA,
        ],
    )
    def k(uid_hbm, utT_hbm, m128_hbm, out_hbm,
          uid_v, slabs, mv, out_v, sema, semb, semm):
        wid = lax.axis_index("s") * _NC + lax.axis_index("c")
        base = wid * _BPW
        mbase = wid * (_BPW * NUM_LATENT // 128)
        pltpu.sync_copy(uid_hbm.at[pl.ds(base, _BPW)],
                        uid_v.at[pl.ds(0, _BPW)])
        mcp = pltpu.async_copy(
            m128_hbm.at[pl.ds(mbase, _BPW * NUM_LATENT // 128), :], mv, semm)
        lanes = lax.iota(jnp.int32, _G)
        l8 = lanes & (_W - 1)

        def fire(w, slot0, sem):
            chunk = uid_v[pl.ds(w * _W, _G)]
            for e in range(_W):
                u = chunk[e]
                col0 = pl.multiple_of((u >> 7) * 128, 128)
                pltpu.async_copy(utT_hbm.at[:, pl.ds(col0, 128)],
                                 slabs.at[slot0 + e], sem)

        def drain(slot0, sem):
            for e in range(_W):
                pltpu.make_async_copy(utT_hbm.at[:, pl.ds(0, 128)],
                                      slabs.at[slot0 + e], sem).wait()

        def compute(w, slot0):
            chunk = uid_v[pl.ds(w * _W, _G)]
            lane = chunk & 127
            sidx = l8 + slot0
            ev = w * _W + l8
            mrow = ev >> 2
            mcol0 = (ev & 3) * NUM_LATENT
            acc = jnp.zeros((_G,), jnp.float32)
            for j in range(NUM_LATENT):
                uj = plsc.load_gather(
                    slabs, [sidx, jnp.full((_G,), j, jnp.int32), lane])
                mj = plsc.load_gather(mv, [mrow, mcol0 + j])
                acc = acc + uj * mj
            out_v[pl.ds(w * _W, _G)] = acc

        fire(0, 0, sema)
        mcp.wait()

        def pair(p, carry):
            fire(2 * p + 1, _W, semb)
            drain(0, sema)
            compute(2 * p, 0)

            @pl.when(p < _NWAVE // 2 - 1)
            def _():
                fire(2 * p + 2, 0, sema)
            drain(_W, semb)
            compute(2 * p + 1, _W)
            return carry

        lax.fori_loop(0, _NWAVE // 2, pair, 0)
        pltpu.sync_copy(out_v.at[pl.ds(0, _BPW)],
                        out_hbm.at[pl.ds(base, _BPW)])

    return k(user_ids, utT, m128)


def kernel(user_ids, movie_ids, user_table, movie_table):
    mrows = _movie_gather(movie_ids.astype(jnp.int32), movie_table)
    m128 = mrows.reshape(BATCH * NUM_LATENT // 128, 128)
    return _user_slab_dot(user_ids.astype(jnp.int32), user_table.T, m128)


# speedup vs baseline: 2.9229x; 1.0067x over previous
import functools

import jax
import jax.numpy as jnp
from jax import lax
from jax.experimental import pallas as pl
from jax.experimental.pallas import tpu as pltpu
from jax.experimental.pallas import tpu_sc as plsc

NUM_LATENT = 32
BATCH = 16384

_info = plsc.get_sparse_core_info()
_NC = _info.num_cores
_NS = _info.num_subcores
_NW = _NC * _NS
_BPW = BATCH // _NW
_CHUNK = 128
_NCHUNK = _BPW // _CHUNK
_G = 16


def _movie_gather(movie_ids, movie_table):
    @functools.partial(
        pl.kernel,
        mesh=plsc.VectorSubcoreMesh(core_axis_name="c", subcore_axis_name="s"),
        out_type=jax.ShapeDtypeStruct((BATCH, NUM_LATENT), jnp.float32),
        compiler_params=pltpu.CompilerParams(needs_layout_passes=False,
                                             use_tc_tiling_on_sc=False),
        scratch_types=[
            pltpu.VMEM((_NCHUNK, _CHUNK), jnp.int32),
            pltpu.VMEM((_BPW, NUM_LATENT), jnp.float32),
            pltpu.SemaphoreType.DMA,
        ],
    )
    def k(mid_hbm, mt_hbm, mrows_hbm, mid_v, mrows_v, sem):
        wid = lax.axis_index("s") * _NC + lax.axis_index("c")
        base = wid * _BPW
        for c in range(_NCHUNK):
            pltpu.sync_copy(mid_hbm.at[pl.ds(base + c * _CHUNK, _CHUNK)],
                            mid_v.at[c])
        copies = []
        for c in range(_NCHUNK):
            copies.append(pltpu.async_copy(
                mt_hbm.at[mid_v.at[c]],
                mrows_v.at[pl.ds(c * _CHUNK, _CHUNK)], sem))
        for cp in copies:
            cp.wait()
        pltpu.sync_copy(mrows_v, mrows_hbm.at[pl.ds(base, _BPW), :])

    return k(movie_ids, movie_table)


_W = 8
_NWAVE = _BPW // _W
_PAD = _BPW + _G


def _user_slab_dot(user_ids, utT, m128):
    @functools.partial(
        pl.kernel,
        mesh=plsc.VectorSubcoreMesh(core_axis_name="c", subcore_axis_name="s"),
        out_type=jax.ShapeDtypeStruct((BATCH,), jnp.float32),
        compiler_params=pltpu.CompilerParams(needs_layout_passes=False),
        scratch_types=[
            pltpu.VMEM((_PAD,), jnp.int32),
            pltpu.VMEM((2 * _W, NUM_LATENT, 128), jnp.float32),
            pltpu.VMEM((_BPW * NUM_LATENT // 128, 128), jnp.float32),
            pltpu.VMEM((_PAD,), jnp.float32),
            pltpu.SemaphoreType.DMA,
            pltpu.SemaphoreType.DMA,
            pltpu.SemaphoreType.DMA,
        ],
    )
    def k(uid_hbm, utT_hbm, m128_hbm, out_hbm,
          uid_v, slabs, mv, out_v, sema, semb, semm):
        wid = lax.axis_index("s") * _NC + lax.axis_index("c")
        base = wid * _BPW
        mbase = wid * (_BPW * NUM_LATENT // 128)
        pltpu.sync_copy(uid_hbm.at[pl.ds(base, _BPW)],
                        uid_v.at[pl.ds(0, _BPW)])
        mcp = pltpu.async_copy(
            m128_hbm.at[pl.ds(mbase, _BPW * NUM_LATENT // 128), :], mv, semm)
        lanes = lax.iota(jnp.int32, _G)
        l8 = lanes & (_W - 1)

        def fire(w, slot0, sem):
            chunk = uid_v[pl.ds(w * _W, _G)]
            for e in range(_W):
                u = chunk[e]
                col0 = pl.multiple_of((u >> 7) * 128, 128)
                pltpu.async_copy(utT_hbm.at[:, pl.ds(col0, 128)],
                                 slabs.at[slot0 + e], sem)

        def drain(slot0, sem):
            for e in range(_W):
                pltpu.make_async_copy(utT_hbm.at[:, pl.ds(0, 128)],
                                      slabs.at[slot0 + e], sem).wait()

        def compute(w, slot0):
            chunk = uid_v[pl.ds(w * _W, _G)]
            lane = chunk & 127
            sidx = l8 + slot0
            ev = w * _W + l8
            mrow = ev >> 2
            mcol0 = (ev & 3) * NUM_LATENT
            acc = jnp.zeros((_G,), jnp.float32)
            for j in range(NUM_LATENT):
                uj = plsc.load_gather(
                    slabs, [sidx, jnp.full((_G,), j, jnp.int32), lane])
                mj = plsc.load_gather(mv, [mrow, mcol0 + j])
                acc = acc + uj * mj
            out_v[pl.ds(w * _W, _G)] = acc

        fire(0, 0, sema)
        mcp.wait()

        def pair(p, carry):
            fire(2 * p + 1, _W, semb)
            drain(0, sema)
            compute(2 * p, 0)

            @pl.when(p < _NWAVE // 2 - 1)
            def _():
                fire(2 * p + 2, 0, sema)
            drain(_W, semb)
            compute(2 * p + 1, _W)
            return carry

        lax.fori_loop(0, _NWAVE // 2, pair, 0)
        pltpu.sync_copy(out_v.at[pl.ds(0, _BPW)],
                        out_hbm.at[pl.ds(base, _BPW)])

    return k(user_ids, utT, m128)


def kernel(user_ids, movie_ids, user_table, movie_table):
    mrows = _movie_gather(movie_ids.astype(jnp.int32), movie_table)
    m128 = mrows.reshape(BATCH * NUM_LATENT // 128, 128)
    return _user_slab_dot(user_ids.astype(jnp.int32), user_table.T, m128)


# speedup vs baseline: 2.9490x; 1.0089x over previous
import functools

import jax
import jax.numpy as jnp
from jax import lax
from jax.experimental import pallas as pl
from jax.experimental.pallas import tpu as pltpu
from jax.experimental.pallas import tpu_sc as plsc

NUM_LATENT = 32
BATCH = 16384
MQROWS = 100000 * NUM_LATENT // 128

_info = plsc.get_sparse_core_info()
_NC = _info.num_cores
_NS = _info.num_subcores
_NW = _NC * _NS
_BPW = BATCH // _NW
_G = 16
_W = 8
_NWAVE = _BPW // _W
_PAD = _BPW + _G


def _embed_dot(user_ids, movie_ids, utT, mt128):
    @functools.partial(
        pl.kernel,
        mesh=plsc.VectorSubcoreMesh(core_axis_name="c", subcore_axis_name="s"),
        out_type=jax.ShapeDtypeStruct((BATCH,), jnp.float32),
        compiler_params=pltpu.CompilerParams(needs_layout_passes=False),
        scratch_types=[
            pltpu.VMEM((_PAD,), jnp.int32),
            pltpu.VMEM((_PAD,), jnp.int32),
            pltpu.VMEM((_BPW,), jnp.int32),
            pltpu.VMEM((2 * _W, NUM_LATENT, 128), jnp.float32),
            pltpu.VMEM((_W, 128), jnp.float32),
            pltpu.VMEM((_W, 128), jnp.float32),
            pltpu.VMEM((_PAD,), jnp.float32),
            pltpu.SemaphoreType.DMA,
            pltpu.SemaphoreType.DMA,
        ],
    )
    def k(uid_hbm, mid_hbm, utT_hbm, mt_hbm, out_hbm,
          uid_v, mid_v, mq_v, slabs, mbufa, mbufb, out_v, sema, semb):
        wid = lax.axis_index("s") * _NC + lax.axis_index("c")
        base = wid * _BPW
        pltpu.sync_copy(uid_hbm.at[pl.ds(base, _BPW)],
                        uid_v.at[pl.ds(0, _BPW)])
        pltpu.sync_copy(mid_hbm.at[pl.ds(base, _BPW)],
                        mid_v.at[pl.ds(0, _BPW)])

        def quads(c, carry):
            mq_v[pl.ds(c * _G, _G)] = mid_v[pl.ds(c * _G, _G)] >> 2
            return carry
        lax.fori_loop(0, _BPW // _G, quads, 0)

        lanes = lax.iota(jnp.int32, _G)
        l8 = lanes & (_W - 1)

        def fire(w, slot0, mbuf, sem):
            chunk = uid_v[pl.ds(w * _W, _G)]
            for e in range(_W):
                u = chunk[e]
                col0 = pl.multiple_of((u >> 7) * 128, 128)
                pltpu.async_copy(utT_hbm.at[:, pl.ds(col0, 128)],
                                 slabs.at[slot0 + e], sem)
            pltpu.async_copy(mt_hbm.at[mq_v.at[pl.ds(w * _W, _W)]], mbuf, sem)

        def drain(slot0, mbuf, sem):
            for e in range(_W):
                pltpu.make_async_copy(utT_hbm.at[:, pl.ds(0, 128)],
                                      slabs.at[slot0 + e], sem).wait()
            pltpu.make_async_copy(mt_hbm.at[pl.ds(0, _W)], mbuf, sem).wait()

        def compute(w, slot0, mbuf):
            uchunk = uid_v[pl.ds(w * _W, _G)]
            mchunk = mid_v[pl.ds(w * _W, _G)]
            ulane = uchunk & 127
            sidx = l8 + slot0
            mcol0 = (mchunk & 3) * NUM_LATENT
            acc = jnp.zeros((_G,), jnp.float32)
            for j in range(NUM_LATENT):
                uj = plsc.load_gather(
                    slabs, [sidx, jnp.full((_G,), j, jnp.int32), ulane])
                mj = plsc.load_gather(mbuf, [l8, mcol0 + j])
                acc = acc + uj * mj
            out_v[pl.ds(w * _W, _G)] = acc

        fire(0, 0, mbufa, sema)

        def pair(p, carry):
            fire(2 * p + 1, _W, mbufb, semb)
            drain(0, mbufa, sema)
            compute(2 * p, 0, mbufa)

            @pl.when(p < _NWAVE // 2 - 1)
            def _():
                fire(2 * p + 2, 0, mbufa, sema)
            drain(_W, mbufb, semb)
            compute(2 * p + 1, _W, mbufb)
            return carry

        lax.fori_loop(0, _NWAVE // 2, pair, 0)
        pltpu.sync_copy(out_v.at[pl.ds(0, _BPW)],
                        out_hbm.at[pl.ds(base, _BPW)])

    return k(user_ids, movie_ids, utT, mt128)


def kernel(user_ids, movie_ids, user_table, movie_table):
    mt128 = movie_table.reshape(MQROWS, 128)
    return _embed_dot(user_ids.astype(jnp.int32),
                      movie_ids.astype(jnp.int32),
                      user_table.T, mt128)
